# baseline (speedup 1.0000x reference)
import functools

import jax
import jax.numpy as jnp
from jax import lax
from jax.experimental import pallas as pl
from jax.experimental.pallas import tpu as pltpu
from jax.experimental.pallas import tpu_sc as plsc

B = 16384
FEAT = 32
MLP_DIM = 128

_NC = 2
_NS = 16
_NW = _NC * _NS
_BPW = B // _NW
_CHUNK = 128
_NCHUNK = _BPW // _CHUNK
_GRP = 16
_CGRP = 8


def _sc_gather_body(cust_hbm, prod_hbm, tcgT_hbm, tpg_hbm, tcm_hbm, tpm_hbm,
                    out_cg, out_pg, out_cm, out_pm,
                    cidx, pidx, cg_v, cm_v, pm_v, colbuf0, colbuf1,
                    sem, sem_g, sem_c0, sem_c1):
    wid = lax.axis_index("s") * _NC + lax.axis_index("c")
    base = wid * _BPW
    kv0 = lax.iota(jnp.int32, 16)
    slots = ((colbuf0, sem_c0), (colbuf1, sem_c1))
    ngrp = _CHUNK // _GRP

    def _fire(cv, half, slot):
        colbuf, sem_c = slots[slot]
        for j in range(_CGRP):
            lane = half * _CGRP + j
            colstart = pl.multiple_of((cv[lane] >> 7) << 7, 128)
            pltpu.async_copy(tcgT_hbm.at[:, pl.ds(colstart, 128)],
                             colbuf.at[pl.ds(j * FEAT, FEAT)], sem_c)

    def _drain_extract(cv, g, half, slot):
        colbuf, sem_c = slots[slot]
        pltpu.make_async_copy(out_cm.at[pl.ds(0, _CGRP * FEAT)],
                              colbuf, sem_c).wait()
        for j in range(_CGRP):
            lane = half * _CGRP + j
            rv = jnp.full((16,), cv[lane] & 127, jnp.int32)
            lo = plsc.load_gather(colbuf, [kv0 + j * FEAT, rv])
            hi = plsc.load_gather(colbuf, [kv0 + (j * FEAT + 16), rv])
            row = g * _GRP + lane
            cg_v[row, pl.ds(0, 16)] = lo
            cg_v[row, pl.ds(16, 16)] = hi

    for c in range(_NCHUNK):
        off = base + c * _CHUNK
        pltpu.sync_copy(cust_hbm.at[pl.ds(off, _CHUNK)], cidx)
        pltpu.sync_copy(prod_hbm.at[pl.ds(off, _CHUNK)], pidx)
        c3 = pltpu.async_copy(tcm_hbm.at[cidx], cm_v, sem)
        c4 = pltpu.async_copy(tpm_hbm.at[pidx], pm_v, sem)

        def _grp_prod(g, _):
            gb = g * _GRP
            pv = pidx[pl.ds(gb, _GRP)]
            for j in range(_GRP):
                pltpu.async_copy(tpg_hbm.at[pv[j]], out_pg.at[off + gb + j],
                                 sem_g)
            return ()

        lax.fori_loop(0, ngrp, _grp_prod, ())

        cv0 = cidx[pl.ds(0, _GRP)]
        _fire(cv0, 0, 0)
        _fire(cv0, 1, 1)

        def _grp_cust(g, cv):
            cvn = cidx[pl.ds((g + 1) * _GRP, _GRP)]
            _drain_extract(cv, g, 0, 0)
            _fire(cvn, 0, 0)
            _drain_extract(cv, g, 1, 1)
            _fire(cvn, 1, 1)
            return cvn

        cv_last = lax.fori_loop(0, ngrp - 1, _grp_cust, cv0)
        _drain_extract(cv_last, ngrp - 1, 0, 0)
        _drain_extract(cv_last, ngrp - 1, 1, 1)

        pltpu.make_async_copy(tpg_hbm.at[pl.ds(0, _CHUNK)],
                              out_pg.at[pl.ds(off, _CHUNK)], sem_g).wait()
        c3.wait()
        c4.wait()
        pltpu.sync_copy(cg_v, out_cg.at[pl.ds(off, _CHUNK)])
        pltpu.sync_copy(cm_v, out_cm.at[pl.ds(off, _CHUNK)])
        pltpu.sync_copy(pm_v, out_pm.at[pl.ds(off, _CHUNK)])


_sc_gather = functools.partial(
    pl.kernel,
    mesh=plsc.VectorSubcoreMesh(core_axis_name="c", subcore_axis_name="s"),
    out_type=[
        jax.ShapeDtypeStruct((B, FEAT), jnp.float32),
        jax.ShapeDtypeStruct((B, FEAT), jnp.float32),
        jax.ShapeDtypeStruct((B, MLP_DIM), jnp.float32),
        jax.ShapeDtypeStruct((B, MLP_DIM), jnp.float32),
    ],
    scratch_types=[
        pltpu.VMEM((_CHUNK,), jnp.int32),
        pltpu.VMEM((_CHUNK,), jnp.int32),
        pltpu.VMEM((_CHUNK, FEAT), jnp.float32),
        pltpu.VMEM((_CHUNK, MLP_DIM), jnp.float32),
        pltpu.VMEM((_CHUNK, MLP_DIM), jnp.float32),
        pltpu.VMEM((_CGRP * FEAT, 128), jnp.float32),
        pltpu.VMEM((_CGRP * FEAT, 128), jnp.float32),
        pltpu.SemaphoreType.DMA,
        pltpu.SemaphoreType.DMA,
        pltpu.SemaphoreType.DMA,
        pltpu.SemaphoreType.DMA,
    ],
    compiler_params=pltpu.CompilerParams(needs_layout_passes=False),
)(_sc_gather_body)


_BLK = 2048


def _tc_body(cg, pg, cm, pm, w0a, w0b, b0, w1, b1, w2, b2, wna, wnb, bn, out):
    h = jnp.dot(cm[...], w0a[...], preferred_element_type=jnp.float32)
    h = h + jnp.dot(pm[...], w0b[...], preferred_element_type=jnp.float32)
    h = jnp.maximum(h + b0[...], 0.0)
    h = jnp.maximum(jnp.dot(h, w1[...], preferred_element_type=jnp.float32) + b1[...], 0.0)
    h = jnp.maximum(jnp.dot(h, w2[...], preferred_element_type=jnp.float32) + b2[...], 0.0)
    g = cg[...] * pg[...]
    out[...] = (jnp.dot(g, wna[...], preferred_element_type=jnp.float32)
                + jnp.dot(h, wnb[...], preferred_element_type=jnp.float32)
                + bn[...])


def _full(shape):
    return pl.BlockSpec(shape, lambda i: (0, 0))


def _tc_dense(cg, pg, cm, pm, w0a, w0b, b0, w1, b1, w2, b2, wna, wnb, bn):
    return pl.pallas_call(
        _tc_body,
        grid=(B // _BLK,),
        in_specs=[
            pl.BlockSpec((_BLK, FEAT), lambda i: (i, 0)),
            pl.BlockSpec((_BLK, FEAT), lambda i: (i, 0)),
            pl.BlockSpec((_BLK, MLP_DIM), lambda i: (i, 0)),
            pl.BlockSpec((_BLK, MLP_DIM), lambda i: (i, 0)),
            _full((MLP_DIM, MLP_DIM)),
            _full((MLP_DIM, MLP_DIM)),
            _full((1, MLP_DIM)),
            _full((MLP_DIM, 64)),
            _full((1, 64)),
            _full((64, FEAT)),
            _full((1, FEAT)),
            _full((FEAT, 1)),
            _full((FEAT, 1)),
            _full((1, 1)),
        ],
        out_specs=pl.BlockSpec((_BLK, 1), lambda i: (i, 0)),
        out_shape=jax.ShapeDtypeStruct((B, 1), jnp.float32),
    )(cg, pg, cm, pm, w0a, w0b, b0, w1, b1, w2, b2, wna, wnb, bn)


def kernel(cust, prod, tab_cust_gmf, tab_prod_gmf, tab_cust_mlp, tab_prod_mlp,
           W0, b0, W1, b1, W2, b2, Wn, bn):
    cust = cust.astype(jnp.int32)
    prod = prod.astype(jnp.int32)
    cg, pg, cm, pm = _sc_gather(cust, prod, tab_cust_gmf.T, tab_prod_gmf,
                                tab_cust_mlp, tab_prod_mlp)
    out = _tc_dense(
        cg, pg, cm, pm,
        W0[:MLP_DIM], W0[MLP_DIM:], b0.reshape(1, -1),
        W1, b1.reshape(1, -1),
        W2, b2.reshape(1, -1),
        Wn[:FEAT], Wn[FEAT:], bn.reshape(1, -1),
    )
    return out.reshape(-1)


# speedup vs baseline: 1.6038x; 1.6038x over previous
import functools

import jax
import jax.numpy as jnp
from jax import lax
from jax.experimental import pallas as pl
from jax.experimental.pallas import tpu as pltpu
from jax.experimental.pallas import tpu_sc as plsc

B = 16384
FEAT = 32
MLP_DIM = 128

_NC = 2
_NS = 16
_NW = _NC * _NS
_BPW = B // _NW
_CHUNK = 64
_NCHUNK = _BPW // _CHUNK
_GRP = 16
_CGRP = 8


def _sc_gather_body(cust_hbm, prod_hbm, tcgT_hbm, tpg_hbm, tcm_hbm, tpm_hbm,
                    out_cg, out_pg, out_cm, out_pm,
                    cidx, pidx, cg_v, pg_v, cm_v, pm_v, colbuf0, colbuf1,
                    sem, sem_g, sem_c0, sem_c1):
    wid = lax.axis_index("s") * _NC + lax.axis_index("c")
    base = wid * _BPW
    kv0 = lax.iota(jnp.int32, 16)
    slots = ((colbuf0, sem_c0), (colbuf1, sem_c1))
    ngrp = _CHUNK // _GRP

    def _fire(cv, half, slot):
        colbuf, sem_c = slots[slot]
        for j in range(_CGRP):
            lane = half * _CGRP + j
            colstart = pl.multiple_of((cv[lane] >> 7) << 7, 128)
            pltpu.async_copy(tcgT_hbm.at[:, pl.ds(colstart, 128)],
                             colbuf.at[pl.ds(j * FEAT, FEAT)], sem_c)

    def _drain_extract(cv, g, half, slot):
        colbuf, sem_c = slots[slot]
        pltpu.make_async_copy(out_cm.at[pl.ds(0, _CGRP * FEAT)],
                              colbuf, sem_c).wait()
        for j in range(_CGRP):
            lane = half * _CGRP + j
            rv = jnp.full((16,), cv[lane] & 127, jnp.int32)
            lo = plsc.load_gather(colbuf, [kv0 + j * FEAT, rv])
            hi = plsc.load_gather(colbuf, [kv0 + (j * FEAT + 16), rv])
            row = g * _GRP + lane
            cg_v[row, pl.ds(0, 16)] = lo
            cg_v[row, pl.ds(16, 16)] = hi

    for c in range(_NCHUNK):
        off = base + c * _CHUNK
        pltpu.sync_copy(cust_hbm.at[pl.ds(off, _CHUNK)], cidx)
        pltpu.sync_copy(prod_hbm.at[pl.ds(off, _CHUNK)], pidx)
        c3 = pltpu.async_copy(tcm_hbm.at[cidx], cm_v, sem)
        c4 = pltpu.async_copy(tpm_hbm.at[pidx], pm_v, sem)

        def _grp_prod(g, _):
            gb = g * _GRP
            pv = pidx[pl.ds(gb, _GRP)]
            for j in range(_GRP):
                pltpu.async_copy(tpg_hbm.at[pv[j]], pg_v.at[gb + j], sem_g)
            return ()

        lax.fori_loop(0, ngrp, _grp_prod, ())

        cv0 = cidx[pl.ds(0, _GRP)]
        _fire(cv0, 0, 0)
        _fire(cv0, 1, 1)

        def _grp_cust(g, cv):
            cvn = cidx[pl.ds((g + 1) * _GRP, _GRP)]
            _drain_extract(cv, g, 0, 0)
            _fire(cvn, 0, 0)
            _drain_extract(cv, g, 1, 1)
            _fire(cvn, 1, 1)
            return cvn

        cv_last = lax.fori_loop(0, ngrp - 1, _grp_cust, cv0)
        _drain_extract(cv_last, ngrp - 1, 0, 0)
        _drain_extract(cv_last, ngrp - 1, 1, 1)

        pltpu.make_async_copy(tpg_hbm.at[pl.ds(0, _CHUNK)], pg_v, sem_g).wait()
        c3.wait()
        c4.wait()
        pltpu.sync_copy(cg_v, out_cg.at[pl.ds(off, _CHUNK)])
        pltpu.sync_copy(pg_v, out_pg.at[pl.ds(off, _CHUNK)])
        pltpu.sync_copy(cm_v, out_cm.at[pl.ds(off, _CHUNK)])
        pltpu.sync_copy(pm_v, out_pm.at[pl.ds(off, _CHUNK)])


_sc_gather = functools.partial(
    pl.kernel,
    mesh=plsc.VectorSubcoreMesh(core_axis_name="c", subcore_axis_name="s"),
    out_type=[
        jax.ShapeDtypeStruct((B, FEAT), jnp.float32),
        jax.ShapeDtypeStruct((B, FEAT), jnp.float32),
        jax.ShapeDtypeStruct((B, MLP_DIM), jnp.float32),
        jax.ShapeDtypeStruct((B, MLP_DIM), jnp.float32),
    ],
    scratch_types=[
        pltpu.VMEM((_CHUNK,), jnp.int32),
        pltpu.VMEM((_CHUNK,), jnp.int32),
        pltpu.VMEM((_CHUNK, FEAT), jnp.float32),
        pltpu.VMEM((_CHUNK, FEAT), jnp.float32),
        pltpu.VMEM((_CHUNK, MLP_DIM), jnp.float32),
        pltpu.VMEM((_CHUNK, MLP_DIM), jnp.float32),
        pltpu.VMEM((_CGRP * FEAT, 128), jnp.float32),
        pltpu.VMEM((_CGRP * FEAT, 128), jnp.float32),
        pltpu.SemaphoreType.DMA,
        pltpu.SemaphoreType.DMA,
        pltpu.SemaphoreType.DMA,
        pltpu.SemaphoreType.DMA,
    ],
    compiler_params=pltpu.CompilerParams(needs_layout_passes=False),
)(_sc_gather_body)


_BLK = 2048


def _tc_body(cg, pg, cm, pm, w0a, w0b, b0, w1, b1, w2, b2, wna, wnb, bn, out):
    h = jnp.dot(cm[...], w0a[...], preferred_element_type=jnp.float32)
    h = h + jnp.dot(pm[...], w0b[...], preferred_element_type=jnp.float32)
    h = jnp.maximum(h + b0[...], 0.0)
    h = jnp.maximum(jnp.dot(h, w1[...], preferred_element_type=jnp.float32) + b1[...], 0.0)
    h = jnp.maximum(jnp.dot(h, w2[...], preferred_element_type=jnp.float32) + b2[...], 0.0)
    g = cg[...] * pg[...]
    out[...] = (jnp.dot(g, wna[...], preferred_element_type=jnp.float32)
                + jnp.dot(h, wnb[...], preferred_element_type=jnp.float32)
                + bn[...])


def _full(shape):
    return pl.BlockSpec(shape, lambda i: (0, 0))


def _tc_dense(cg, pg, cm, pm, w0a, w0b, b0, w1, b1, w2, b2, wna, wnb, bn):
    return pl.pallas_call(
        _tc_body,
        grid=(B // _BLK,),
        in_specs=[
            pl.BlockSpec((_BLK, FEAT), lambda i: (i, 0)),
            pl.BlockSpec((_BLK, FEAT), lambda i: (i, 0)),
            pl.BlockSpec((_BLK, MLP_DIM), lambda i: (i, 0)),
            pl.BlockSpec((_BLK, MLP_DIM), lambda i: (i, 0)),
            _full((MLP_DIM, MLP_DIM)),
            _full((MLP_DIM, MLP_DIM)),
            _full((1, MLP_DIM)),
            _full((MLP_DIM, 64)),
            _full((1, 64)),
            _full((64, FEAT)),
            _full((1, FEAT)),
            _full((FEAT, 1)),
            _full((FEAT, 1)),
            _full((1, 1)),
        ],
        out_specs=pl.BlockSpec((_BLK, 1), lambda i: (i, 0)),
        out_shape=jax.ShapeDtypeStruct((B, 1), jnp.float32),
    )(cg, pg, cm, pm, w0a, w0b, b0, w1, b1, w2, b2, wna, wnb, bn)


def kernel(cust, prod, tab_cust_gmf, tab_prod_gmf, tab_cust_mlp, tab_prod_mlp,
           W0, b0, W1, b1, W2, b2, Wn, bn):
    cust = cust.astype(jnp.int32)
    prod = prod.astype(jnp.int32)
    cg, pg, cm, pm = _sc_gather(cust, prod, tab_cust_gmf.T, tab_prod_gmf,
                                tab_cust_mlp, tab_prod_mlp)
    out = _tc_dense(
        cg, pg, cm, pm,
        W0[:MLP_DIM], W0[MLP_DIM:], b0.reshape(1, -1),
        W1, b1.reshape(1, -1),
        W2, b2.reshape(1, -1),
        Wn[:FEAT], Wn[FEAT:], bn.reshape(1, -1),
    )
    return out.reshape(-1)
